# baseline (speedup 1.0000x reference)
import functools

import jax
import jax.numpy as jnp
from jax import lax
from jax.experimental import pallas as pl
from jax.experimental.pallas import tpu as pltpu

_BB = 8


def _body(idx_ref, cqt_ref, pos_ref, out_ref, pbuf, sem):
    g = pl.program_id(0)
    for i in range(_BB):
        b = g * _BB + i
        sel = idx_ref[b]
        pltpu.make_async_copy(pos_ref.at[b, sel], pbuf.at[i], sem).start()
    out_ref[...] = cqt_ref[...]
    for i in range(_BB):
        b = g * _BB + i
        sel = idx_ref[b]
        pltpu.make_async_copy(pos_ref.at[b, sel], pbuf.at[i], sem).wait()
        out_ref[i, pl.ds(sel, 1)] = pbuf[i][None]


def kernel(cqt, pos_cqt, channel_idx):
    B, C, F, T = cqt.shape
    idx = channel_idx.astype(jnp.int32)

    grid_spec = pltpu.PrefetchScalarGridSpec(
        num_scalar_prefetch=1,
        grid=(B // _BB,),
        in_specs=[
            pl.BlockSpec((_BB, C, F, T), lambda g, idx_ref: (g, 0, 0, 0)),
            pl.BlockSpec(memory_space=pltpu.MemorySpace.HBM),
        ],
        out_specs=pl.BlockSpec((_BB, C, F, T), lambda g, idx_ref: (g, 0, 0, 0)),
        scratch_shapes=[
            pltpu.MemorySpace.VMEM((_BB, F, T), jnp.float32),
            pltpu.SemaphoreType.DMA,
        ],
    )
    return pl.pallas_call(
        _body,
        grid_spec=grid_spec,
        out_shape=jax.ShapeDtypeStruct(cqt.shape, cqt.dtype),
    )(idx, cqt, pos_cqt)
